# baseline (speedup 1.0000x reference)
import functools

import jax
import jax.numpy as jnp
from jax import lax
from jax.experimental import pallas as pl
from jax.experimental.pallas import tpu as pltpu
from jax.experimental.pallas import tpu_sc as plsc


def _make_table_prep(vocab: int, dim: int, pitch: int, vblk: int):
  grid = (vocab + vblk - 1) // vblk

  def prep_kernel(xt_ref, o_ref):
    xt = xt_ref[...]
    o_ref[...] = jnp.concatenate(
        [xt.T, jnp.zeros((vblk, pitch - dim), jnp.float32)], axis=1)

  return pl.pallas_call(
      prep_kernel,
      grid=(grid,),
      in_specs=[pl.BlockSpec((dim, vblk), lambda i: (0, i))],
      out_specs=pl.BlockSpec((vblk, pitch), lambda i: (i, 0)),
      out_shape=jax.ShapeDtypeStruct((vocab, pitch), jnp.float32),
  )


def _make_gather(batch: int, hist: int, dim: int, num_workers: int,
                 num_cores: int, chunk: int, nbuf: int):
  n_total = batch * hist
  n_per_w = n_total // num_workers
  n_chunks = n_per_w // chunk
  assert n_per_w % chunk == 0 and n_chunks % nbuf == 0 and chunk % 8 == 0
  n_outer = n_chunks // nbuf
  mesh = plsc.VectorSubcoreMesh(core_axis_name="c", subcore_axis_name="s")

  @functools.partial(
      pl.kernel,
      out_type=jax.ShapeDtypeStruct((n_total, dim), jnp.float32),
      mesh=mesh,
      scratch_types=[
          pltpu.VMEM((nbuf, chunk), jnp.int32),
          pltpu.VMEM((nbuf, chunk, dim), jnp.float32),
      ] + [pltpu.SemaphoreType.DMA] * (2 * nbuf),
      compiler_params=pltpu.CompilerParams(use_tc_tiling_on_sc=False),
  )
  def gather_kernel(idx_hbm, table_hbm, out_hbm, idx_v, rows_v, *sems):
    gsems, wsems = sems[:nbuf], sems[nbuf:]
    wid = lax.axis_index("s") * num_cores + lax.axis_index("c")
    base = pl.multiple_of(wid * n_per_w, 8)

    def start(g, b):
      off = pl.multiple_of(base + g * chunk, 8)
      pltpu.sync_copy(idx_hbm.at[pl.ds(off, chunk)], idx_v.at[b])
      pltpu.async_copy(table_hbm.at[idx_v.at[b]], rows_v.at[b], gsems[b])

    def wait_gather(b):
      pltpu.make_async_copy(table_hbm.at[idx_v.at[b]], rows_v.at[b],
                            gsems[b]).wait()

    def start_write(g, b):
      off = pl.multiple_of(base + g * chunk, 8)
      pltpu.async_copy(rows_v.at[b], out_hbm.at[pl.ds(off, chunk)],
                       wsems[b])

    def wait_write(g, b):
      off = pl.multiple_of(base + g * chunk, 8)
      pltpu.make_async_copy(rows_v.at[b], out_hbm.at[pl.ds(off, chunk)],
                            wsems[b]).wait()

    for b in range(nbuf):
      start(b, b)

    def body(t, carry):
      g0 = t * nbuf
      for b in range(nbuf):
        p = g0 + b
        wait_gather(b)
        start_write(p, b)
        nxt = p + nbuf

        @pl.when(nxt < n_chunks)
        def _refill():
          wait_write(p, b)
          start(nxt, b)
      return carry

    lax.fori_loop(0, n_outer, body, 0)

    for b in range(nbuf):
      wait_write(n_chunks - nbuf + b, b)

  return gather_kernel


def _make_out_retile(batch: int, hist: int, dim: int, bblk: int):
  assert batch % bblk == 0 and hist % 2 == 0
  hp = hist // 2
  rblk = bblk * hp

  def retile_kernel(x_ref, o_ref):
    x3 = x_ref[...].reshape(bblk, hp, 2 * dim)
    for k in range(hp):
      slab_t = x3[:, k, :].T
      o_ref[2 * k] = slab_t[:dim]
      o_ref[2 * k + 1] = slab_t[dim:]

  return pl.pallas_call(
      retile_kernel,
      grid=(batch // bblk,),
      in_specs=[pl.BlockSpec((rblk, 2 * dim), lambda i: (i, 0))],
      out_specs=pl.BlockSpec((hist, dim, bblk), lambda i: (0, 0, i)),
      out_shape=jax.ShapeDtypeStruct((hist, dim, batch), jnp.float32),
  )


def kernel(indices, embeddings):
  batch, hist = indices.shape
  vocab, dim = embeddings.shape
  pitch = 128
  info = plsc.get_sparse_core_info()
  num_workers = info.num_cores * info.num_subcores

  prep = _make_table_prep(vocab, dim, pitch, vblk=512)
  table128 = prep(embeddings.T)
  table2 = table128.reshape(-1, dim)
  idx2 = indices.reshape(batch * hist) * 2

  gather = _make_gather(batch, hist, dim, num_workers, info.num_cores,
                        chunk=400, nbuf=4)
  rows = gather(idx2, table2)

  retile = _make_out_retile(batch, hist, dim, bblk=256)
  out_t = retile(rows.reshape(-1, 2 * dim))
  return jnp.transpose(out_t, (2, 0, 1))


# speedup vs baseline: 1.7606x; 1.7606x over previous
import functools

import jax
import jax.numpy as jnp
from jax import lax
from jax.experimental import pallas as pl
from jax.experimental.pallas import tpu as pltpu
from jax.experimental.pallas import tpu_sc as plsc


def _make_gather(batch: int, hist: int, dim: int, num_workers: int,
                 num_cores: int, chunk: int, nbuf: int):
  n_total = batch * hist
  n_per_w = n_total // num_workers
  n_chunks = n_per_w // chunk
  assert n_per_w % chunk == 0 and n_chunks % nbuf == 0 and chunk % 8 == 0
  n_outer = n_chunks // nbuf
  mesh = plsc.VectorSubcoreMesh(core_axis_name="c", subcore_axis_name="s")

  @functools.partial(
      pl.kernel,
      out_type=jax.ShapeDtypeStruct((n_total, dim), jnp.float32),
      mesh=mesh,
      scratch_types=[
          pltpu.VMEM((nbuf, chunk), jnp.int32),
          pltpu.VMEM((nbuf, chunk, dim), jnp.float32),
      ] + [pltpu.SemaphoreType.DMA] * (2 * nbuf),
      compiler_params=pltpu.CompilerParams(use_tc_tiling_on_sc=False),
  )
  def gather_kernel(idx_hbm, table_hbm, out_hbm, idx_v, rows_v, *sems):
    gsems, wsems = sems[:nbuf], sems[nbuf:]
    wid = lax.axis_index("s") * num_cores + lax.axis_index("c")
    base = pl.multiple_of(wid * n_per_w, 8)

    def start(g, b):
      off = pl.multiple_of(base + g * chunk, 8)
      pltpu.sync_copy(idx_hbm.at[pl.ds(off, chunk)], idx_v.at[b])
      pltpu.async_copy(table_hbm.at[idx_v.at[b]], rows_v.at[b], gsems[b])

    def wait_gather(b):
      pltpu.make_async_copy(table_hbm.at[idx_v.at[b]], rows_v.at[b],
                            gsems[b]).wait()

    def start_write(g, b):
      off = pl.multiple_of(base + g * chunk, 8)
      pltpu.async_copy(rows_v.at[b], out_hbm.at[pl.ds(off, chunk)],
                       wsems[b])

    def wait_write(g, b):
      off = pl.multiple_of(base + g * chunk, 8)
      pltpu.make_async_copy(rows_v.at[b], out_hbm.at[pl.ds(off, chunk)],
                            wsems[b]).wait()

    for b in range(nbuf):
      start(b, b)

    def body(t, carry):
      g0 = t * nbuf
      for b in range(nbuf):
        p = g0 + b
        wait_gather(b)
        start_write(p, b)
        nxt = p + nbuf

        @pl.when(nxt < n_chunks)
        def _refill():
          wait_write(p, b)
          start(nxt, b)
      return carry

    lax.fori_loop(0, n_outer, body, 0)

    for b in range(nbuf):
      wait_write(n_chunks - nbuf + b, b)

  return gather_kernel


def _make_out_retile(batch: int, hist: int, dim: int, bblk: int):
  assert batch % bblk == 0 and hist % 2 == 0
  hp = hist // 2
  rblk = bblk * hp

  def retile_kernel(x_ref, o_ref):
    x3 = x_ref[...].reshape(bblk, hp, 2 * dim)
    for k in range(hp):
      slab_t = x3[:, k, :].T
      o_ref[2 * k] = slab_t[:dim]
      o_ref[2 * k + 1] = slab_t[dim:]

  return pl.pallas_call(
      retile_kernel,
      grid=(batch // bblk,),
      in_specs=[pl.BlockSpec((rblk, 2 * dim), lambda i: (i, 0))],
      out_specs=pl.BlockSpec((hist, dim, bblk), lambda i: (0, 0, i)),
      out_shape=jax.ShapeDtypeStruct((hist, dim, batch), jnp.float32),
  )


def kernel(indices, embeddings):
  batch, hist = indices.shape
  vocab, dim = embeddings.shape
  pitch = 128
  info = plsc.get_sparse_core_info()
  num_workers = info.num_cores * info.num_subcores

  table2 = jnp.pad(embeddings, ((0, 0), (0, pitch - dim))).reshape(-1, dim)
  idx2 = indices.reshape(batch * hist) * 2

  gather = _make_gather(batch, hist, dim, num_workers, info.num_cores,
                        chunk=400, nbuf=4)
  rows = gather(idx2, table2)

  retile = _make_out_retile(batch, hist, dim, bblk=256)
  out_t = retile(rows.reshape(-1, 2 * dim))
  return jnp.transpose(out_t, (2, 0, 1))


# speedup vs baseline: 1.7912x; 1.0173x over previous
import functools

import jax
import jax.numpy as jnp
from jax import lax
from jax.experimental import pallas as pl
from jax.experimental.pallas import tpu as pltpu
from jax.experimental.pallas import tpu_sc as plsc


def _make_gather(batch: int, hist: int, dim: int, num_workers: int,
                 num_cores: int, chunk: int, nbuf: int):
  n_total = batch * hist
  n_per_w = n_total // num_workers
  n_chunks = n_per_w // chunk
  assert n_per_w % chunk == 0 and n_chunks % nbuf == 0 and chunk % 8 == 0
  n_outer = n_chunks // nbuf
  mesh = plsc.VectorSubcoreMesh(core_axis_name="c", subcore_axis_name="s")

  @functools.partial(
      pl.kernel,
      out_type=jax.ShapeDtypeStruct((n_total, dim), jnp.float32),
      mesh=mesh,
      scratch_types=[
          pltpu.VMEM((nbuf, chunk), jnp.int32),
          pltpu.VMEM((nbuf, chunk, dim), jnp.float32),
      ] + [pltpu.SemaphoreType.DMA] * (2 * nbuf),
      compiler_params=pltpu.CompilerParams(use_tc_tiling_on_sc=False),
  )
  def gather_kernel(idx_hbm, table_hbm, out_hbm, idx_v, rows_v, *sems):
    gsems, wsems = sems[:nbuf], sems[nbuf:]
    wid = lax.axis_index("s") * num_cores + lax.axis_index("c")
    base = pl.multiple_of(wid * n_per_w, 8)

    def start(g, b):
      off = pl.multiple_of(base + g * chunk, 8)
      pltpu.sync_copy(idx_hbm.at[pl.ds(off, chunk)], idx_v.at[b])
      pltpu.async_copy(table_hbm.at[idx_v.at[b]], rows_v.at[b], gsems[b])

    def wait_gather(b):
      pltpu.make_async_copy(table_hbm.at[idx_v.at[b]], rows_v.at[b],
                            gsems[b]).wait()

    def start_write(g, b):
      off = pl.multiple_of(base + g * chunk, 8)
      pltpu.async_copy(rows_v.at[b], out_hbm.at[pl.ds(off, chunk)],
                       wsems[b])

    def wait_write(g, b):
      off = pl.multiple_of(base + g * chunk, 8)
      pltpu.make_async_copy(rows_v.at[b], out_hbm.at[pl.ds(off, chunk)],
                            wsems[b]).wait()

    for b in range(nbuf):
      start(b, b)

    def body(t, carry):
      g0 = t * nbuf
      for b in range(nbuf):
        p = g0 + b
        wait_gather(b)
        start_write(p, b)
        nxt = p + nbuf

        @pl.when(nxt < n_chunks)
        def _refill():
          wait_write(p, b)
          start(nxt, b)
      return carry

    lax.fori_loop(0, n_outer, body, 0)

    for b in range(nbuf):
      wait_write(n_chunks - nbuf + b, b)

  return gather_kernel


def _make_out_retile(batch: int, hist: int, dim: int, bblk: int):
  assert batch % bblk == 0 and hist % 2 == 0
  hp = hist // 2
  rblk = bblk * hp

  def retile_kernel(x_ref, o_ref):
    x3 = x_ref[...].reshape(bblk, hp, 2 * dim)
    for k in range(hp):
      slab_t = x3[:, k, :].T
      o_ref[2 * k] = slab_t[:dim]
      o_ref[2 * k + 1] = slab_t[dim:]

  return pl.pallas_call(
      retile_kernel,
      grid=(batch // bblk,),
      in_specs=[pl.BlockSpec((rblk, 2 * dim), lambda i: (i, 0))],
      out_specs=pl.BlockSpec((hist, dim, bblk), lambda i: (0, 0, i)),
      out_shape=jax.ShapeDtypeStruct((hist, dim, batch), jnp.float32),
  )


def kernel(indices, embeddings):
  batch, hist = indices.shape
  vocab, dim = embeddings.shape
  pitch = 128
  info = plsc.get_sparse_core_info()
  num_workers = info.num_cores * info.num_subcores

  table2 = jnp.pad(embeddings, ((0, 0), (0, pitch - dim))).reshape(-1, dim)
  idx2 = indices.reshape(batch * hist) * 2

  gather = _make_gather(batch, hist, dim, num_workers, info.num_cores,
                        chunk=400, nbuf=4)
  rows = gather(idx2, table2)

  retile = _make_out_retile(batch, hist, dim, bblk=512)
  out_t = retile(rows.reshape(-1, 2 * dim))
  return jnp.transpose(out_t, (2, 0, 1))
